# baseline (speedup 1.0000x reference)
import jax
import jax.numpy as jnp
import numpy as np
from jax import lax
from jax.experimental import pallas as pl

_K = 64
_N = 32768
_ROWS = 128
_INT32_MIN = np.int32(-2147483648)


from jax.experimental.pallas import tpu as pltpu
from jax.experimental.pallas import tpu_sc as plsc

_NC = 2
_NS = 16
_NW = _NC * _NS
_L = 16
_RPW = _ROWS // _NW
_NV = _N // _L
_NSETS = 8
_BV = 8
_NB = _NV // _BV
_VPB = _NB // _NSETS
_CAP = 1024
_MANT = np.int32(0x7FFFFFFF)


def _sv(v):
    b = lax.bitcast_convert_type(v, jnp.int32)
    return b ^ ((b >> 31) & _MANT)


def _sv_inv_f(sv):
    return lax.bitcast_convert_type(sv ^ ((sv >> 31) & _MANT), jnp.float32)


def _popcnt(mask):
    return plsc.all_reduce_population_count(mask)[0]


def _sc_body(x_hbm, o_hbm, rbuf, zero_buf, smaxs, cidx, kbuf, kidx,
             sin0, sin1, sout):
    wid = lax.axis_index("s") * _NC + lax.axis_index("c")
    sins = [sin0, sin1]
    r0 = wid * _RPW
    pending_in = [None] * _RPW
    pending_in[0] = pltpu.async_copy(
        x_hbm.at[r0], rbuf.at[pl.ds(0, _N)], sins[0])
    iota = lax.broadcasted_iota(jnp.int32, (_L,), 0)
    zvec = jnp.zeros((_L,), jnp.float32)
    ninf = jnp.full((_L,), -jnp.inf, jnp.float32)
    ipad = jnp.full((_L,), 2 * _N, jnp.int32)
    ivzero = jnp.zeros((_L,), jnp.int32)
    imin = jnp.int32(_INT32_MIN)

    def zb(i, c):
        for u in range(8):
            zero_buf[pl.ds(i * 8 * _L + u * _L, _L)] = zvec
        return c

    lax.fori_loop(0, _NV // 8, zb, 0)
    for _u in range(_K // _L):
        kidx[pl.ds(_u * _L, _L)] = ivzero
    rbuf[pl.ds(2 * _N, _L)] = ninf

    def gv(ix):
        return plsc.load_gather(rbuf, [ix])

    def pad(cnt):
        cidx[pl.ds(cnt, _L)] = ipad

    def build_keys(nv):
        def kp(i, c):
            kbuf[pl.ds(i * _L, _L)] = _sv(gv(cidx[pl.ds(i * _L, _L)]))
            return c

        lax.fori_loop(0, nv, kp, 0)

    def buffer_descent(nv):
        def bit_body(bi, p):
            cand = p | lax.shift_left(jnp.int32(1), 31 - bi)
            cand_sv = cand ^ imin

            def cb(i, cv):
                sv = kbuf[pl.ds(i * _L, _L)]
                return cv + (sv >= cand_sv).astype(jnp.int32)

            cv = lax.fori_loop(0, nv, cb, ivzero)
            return jnp.where(jnp.sum(cv) >= _K, cand, p)

        p = lax.fori_loop(0, 32, bit_body, jnp.int32(0))
        return p ^ imin

    def compact(nv, m_sv):
        def cb(i, carry):
            ncnt, eqc = carry
            ix = cidx[pl.ds(i * _L, _L)]
            sv = kbuf[pl.ds(i * _L, _L)]
            gtm = sv > m_sv
            eqm = sv == m_sv
            scan = plsc.cumsum(eqm.astype(jnp.int32))
            keep = gtm | (eqm & ((eqc + scan) <= _K))
            plsc.store_compressed(cidx.at[pl.ds(ncnt, _L)], ix, mask=keep)
            return (ncnt + _popcnt(keep), eqc + _popcnt(eqm))

        ncnt, _ = lax.fori_loop(0, nv, cb, (jnp.int32(0), jnp.int32(0)))
        return ncnt

    def rebuild(cnt):
        pad(cnt)
        nv = (cnt + _L - 1) // _L
        build_keys(nv)
        m_sv = buffer_descent(nv)
        return compact(nv, m_sv), m_sv

    def process_row(base):
        svg = []
        for s in range(_NSETS):
            def sb(h, ms, s=s):
                b = h * 4
                bms = []
                for w in range(4):
                    off = base + (s * _VPB + b + w) * _BV * _L
                    bm = rbuf[pl.ds(off, _L)]
                    for u in range(1, _BV):
                        bm = jnp.maximum(bm, rbuf[pl.ds(off + u * _L, _L)])
                    bms.append(bm)
                reds = [jnp.max(bm) for bm in bms]
                for w in range(4):
                    smaxs[s * _VPB + b + w] = reds[w]
                return jnp.maximum(jnp.maximum(ms, jnp.maximum(bms[0], bms[1])),
                                   jnp.maximum(bms[2], bms[3]))

            mx = lax.fori_loop(0, _VPB // 4, sb, ninf)
            svg.append(_sv(mx))

        def gbit(bi, p):
            cand = p | lax.shift_left(jnp.int32(1), 31 - bi)
            cand_sv = cand ^ imin
            cv = ivzero
            for s in range(_NSETS):
                cv = cv + (svg[s] >= cand_sv).astype(jnp.int32)
            return jnp.where(jnp.sum(cv) >= _K, cand, p)

        m_sv0 = lax.fori_loop(0, 32, gbit, jnp.int32(0)) ^ imin
        m_f0 = _sv_inv_f(m_sv0)

        def coll(b, carry):
            cnt, m_f = carry
            nhit = smaxs[b] >= m_f

            def app(carry):
                cnt, m_f = carry

                def reb(c2):
                    cnt3, m_sv = rebuild(c2[0])
                    return (cnt3, _sv_inv_f(m_sv))

                cnt, m_f = lax.cond(cnt > _CAP - _BV * _L, reb,
                                    lambda c2: c2, (cnt, m_f))
                msks, ncs = [], []
                for u in range(_BV):
                    off = b * _BV * _L + u * _L
                    v = rbuf[pl.ds(base + off, _L)]
                    msks.append(v >= m_f)
                    ncs.append(_popcnt(msks[u]))
                offs = [cnt]
                for u in range(_BV):
                    offs.append(offs[u] + ncs[u])
                for u in range(_BV):
                    off = b * _BV * _L + u * _L
                    plsc.store_compressed(cidx.at[pl.ds(offs[u], _L)],
                                          iota + (base + off), mask=msks[u])
                return (offs[_BV], m_f)

            return lax.cond(nhit, app, lambda c: c, (cnt, m_f))

        cnt, _ = lax.fori_loop(0, _NB, coll, (jnp.int32(0), m_f0))

        pad(cnt)
        nv = (cnt + _L - 1) // _L
        build_keys(nv)
        t_sv = buffer_descent(nv)

        def gcount(i, cv):
            sv = kbuf[pl.ds(i * _L, _L)]
            return cv + (sv > t_sv).astype(jnp.int32)

        need = _K - jnp.sum(lax.fori_loop(0, nv, gcount, ivzero))

        def ibit(bi, cut):
            bit = lax.shift_left(jnp.int32(1), 14 - bi)
            tmp = cut + bit - 1

            def cb(i, cv):
                ix = cidx[pl.ds(i * _L, _L)]
                sv = kbuf[pl.ds(i * _L, _L)]
                hit = (sv == t_sv) & ((ix - base) <= tmp)
                return cv + hit.astype(jnp.int32)

            cv = lax.fori_loop(0, nv, cb, ivzero)
            return jnp.where(jnp.sum(cv) >= need, cut, cut + bit)

        idx_star = lax.fori_loop(0, 15, ibit, jnp.int32(0))
        return nv, t_sv, idx_star

    def scatter_row(base, nv, t_sv, idx_star):
        def scat(i, kc):
            ix = cidx[pl.ds(i * _L, _L)]
            v = gv(ix)
            sv = kbuf[pl.ds(i * _L, _L)]
            ixo = ix - base
            keep = (sv > t_sv) | ((sv == t_sv) & (ixo <= idx_star))
            plsc.store_scatter(zero_buf, [ixo], v, mask=keep)
            plsc.store_compressed(kidx.at[pl.ds(kc, _L)], ixo, mask=keep)
            return kc + _popcnt(keep)

        lax.fori_loop(0, nv, scat, jnp.int32(0))

    def unscatter_prev():
        for u in range(_K // _L):
            ixo = kidx[pl.ds(u * _L, _L)]
            plsc.store_scatter(zero_buf, [ixo], zvec)

    out_prev = None
    for rr in range(_RPW):
        base = (rr % 2) * _N
        if rr + 1 < _RPW:
            nxt = (rr + 1) % 2
            pending_in[rr + 1] = pltpu.async_copy(
                x_hbm.at[r0 + rr + 1], rbuf.at[pl.ds(nxt * _N, _N)],
                sins[nxt])
        pending_in[rr].wait()
        nv, t_sv, idx_star = process_row(base)
        if out_prev is not None:
            out_prev.wait()
            unscatter_prev()
        scatter_row(base, nv, t_sv, idx_star)
        out_prev = pltpu.async_copy(zero_buf, o_hbm.at[r0 + rr], sout)
    out_prev.wait()


def _sc_kernel(x):
    f = pl.kernel(
        _sc_body,
        out_type=jax.ShapeDtypeStruct((_ROWS, _N), jnp.float32),
        mesh=plsc.VectorSubcoreMesh(core_axis_name="c", subcore_axis_name="s",
                                    num_cores=_NC, num_subcores=_NS),
        scratch_types=[
            pltpu.VMEM((2 * _N + _L,), jnp.float32),
            pltpu.VMEM((_N,), jnp.float32),
            pltpu.SMEM((_NB,), jnp.float32),
            pltpu.VMEM((_CAP + _L,), jnp.int32),
            pltpu.VMEM((_CAP + _L,), jnp.int32),
            pltpu.VMEM((_K + _L,), jnp.int32),
            pltpu.SemaphoreType.DMA,
            pltpu.SemaphoreType.DMA,
            pltpu.SemaphoreType.DMA,
        ],
        compiler_params=pltpu.CompilerParams(needs_layout_passes=False),
    )
    return f(x)


@jax.jit
def kernel(x):
    return _sc_kernel(x)


# speedup vs baseline: 1.0147x; 1.0147x over previous
import jax
import jax.numpy as jnp
import numpy as np
from jax import lax
from jax.experimental import pallas as pl

_K = 64
_N = 32768
_ROWS = 128
_INT32_MIN = np.int32(-2147483648)


from jax.experimental.pallas import tpu as pltpu
from jax.experimental.pallas import tpu_sc as plsc

_NC = 2
_NS = 16
_NW = _NC * _NS
_L = 16
_RPW = _ROWS // _NW
_NV = _N // _L
_NSETS = 8
_BV = 8
_NB = _NV // _BV
_VPB = _NB // _NSETS
_CAP = 1024
_MANT = np.int32(0x7FFFFFFF)


def _sv(v):
    b = lax.bitcast_convert_type(v, jnp.int32)
    return b ^ ((b >> 31) & _MANT)


def _sv_inv_f(sv):
    return lax.bitcast_convert_type(sv ^ ((sv >> 31) & _MANT), jnp.float32)


def _popcnt(mask):
    return plsc.all_reduce_population_count(mask)[0]


def _sc_body(x_hbm, o_hbm, rbuf, zero_buf, smaxs, cidx, kbuf, kidx,
             sin0, sin1, sout):
    wid = lax.axis_index("s") * _NC + lax.axis_index("c")
    sins = [sin0, sin1]
    r0 = wid * _RPW
    pending_in = [None] * _RPW
    pending_in[0] = pltpu.async_copy(
        x_hbm.at[r0], rbuf.at[pl.ds(0, _N)], sins[0])
    iota = lax.broadcasted_iota(jnp.int32, (_L,), 0)
    zvec = jnp.zeros((_L,), jnp.float32)
    ninf = jnp.full((_L,), -jnp.inf, jnp.float32)
    ipad = jnp.full((_L,), 2 * _N, jnp.int32)
    ivzero = jnp.zeros((_L,), jnp.int32)
    imin = jnp.int32(_INT32_MIN)

    def zb(i, c):
        for u in range(8):
            zero_buf[pl.ds(i * 8 * _L + u * _L, _L)] = zvec
        return c

    lax.fori_loop(0, _NV // 8, zb, 0)
    for _u in range(_K // _L):
        kidx[pl.ds(_u * _L, _L)] = ivzero
    rbuf[pl.ds(2 * _N, _L)] = ninf

    def gv(ix):
        return plsc.load_gather(rbuf, [ix])

    def pad(cnt):
        cidx[pl.ds(cnt, _L)] = ipad

    def build_keys(nv):
        def kp(i, c):
            kbuf[pl.ds(i * _L, _L)] = _sv(gv(cidx[pl.ds(i * _L, _L)]))
            return c

        lax.fori_loop(0, nv, kp, 0)

    def buffer_descent(nv):
        def bit_body(bi, p):
            cand = p | lax.shift_left(jnp.int32(1), 31 - bi)
            cand_sv = cand ^ imin

            def cb(i, cv):
                sv = kbuf[pl.ds(i * _L, _L)]
                return cv + (sv >= cand_sv).astype(jnp.int32)

            cv = lax.fori_loop(0, nv, cb, ivzero)
            return jnp.where(jnp.sum(cv) >= _K, cand, p)

        p = lax.fori_loop(0, 32, bit_body, jnp.int32(0))
        return p ^ imin

    def compact(nv, m_sv):
        def cb(i, carry):
            ncnt, eqc = carry
            ix = cidx[pl.ds(i * _L, _L)]
            sv = kbuf[pl.ds(i * _L, _L)]
            gtm = sv > m_sv
            eqm = sv == m_sv
            scan = plsc.cumsum(eqm.astype(jnp.int32))
            keep = gtm | (eqm & ((eqc + scan) <= _K))
            plsc.store_compressed(cidx.at[pl.ds(ncnt, _L)], ix, mask=keep)
            return (ncnt + _popcnt(keep), eqc + _popcnt(eqm))

        ncnt, _ = lax.fori_loop(0, nv, cb, (jnp.int32(0), jnp.int32(0)))
        return ncnt

    def rebuild(cnt):
        pad(cnt)
        nv = (cnt + _L - 1) // _L
        build_keys(nv)
        m_sv = buffer_descent(nv)
        return compact(nv, m_sv), m_sv

    def process_row(base):
        svg = []
        for s in range(_NSETS):
            def sb(h, ms, s=s):
                b = h * 2
                off = base + (s * _VPB + b) * _BV * _L
                bm0 = rbuf[pl.ds(off, _L)]
                for u in range(1, _BV):
                    bm0 = jnp.maximum(bm0, rbuf[pl.ds(off + u * _L, _L)])
                smaxs[s * _VPB + b] = jnp.max(bm0)
                off1 = off + _BV * _L
                bm1 = rbuf[pl.ds(off1, _L)]
                for u in range(1, _BV):
                    bm1 = jnp.maximum(bm1, rbuf[pl.ds(off1 + u * _L, _L)])
                smaxs[s * _VPB + b + 1] = jnp.max(bm1)
                return jnp.maximum(ms, jnp.maximum(bm0, bm1))

            mx = lax.fori_loop(0, _VPB // 2, sb, ninf)
            svg.append(_sv(mx))

        def gbit(bi, p):
            cand = p | lax.shift_left(jnp.int32(1), 31 - bi)
            cand_sv = cand ^ imin
            cv = ivzero
            for s in range(_NSETS):
                cv = cv + (svg[s] >= cand_sv).astype(jnp.int32)
            return jnp.where(jnp.sum(cv) >= _K, cand, p)

        m_sv0 = lax.fori_loop(0, 32, gbit, jnp.int32(0)) ^ imin
        m_f0 = _sv_inv_f(m_sv0)

        def coll(b, carry):
            cnt, m_f = carry
            nhit = smaxs[b] >= m_f

            def app(carry):
                cnt, m_f = carry

                def reb(c2):
                    cnt3, m_sv = rebuild(c2[0])
                    return (cnt3, _sv_inv_f(m_sv))

                cnt, m_f = lax.cond(cnt > _CAP - _BV * _L, reb,
                                    lambda c2: c2, (cnt, m_f))
                msks, ncs = [], []
                for u in range(_BV):
                    off = b * _BV * _L + u * _L
                    v = rbuf[pl.ds(base + off, _L)]
                    msks.append(v >= m_f)
                    ncs.append(_popcnt(msks[u]))
                offs = [cnt]
                for u in range(_BV):
                    offs.append(offs[u] + ncs[u])
                for u in range(_BV):
                    off = b * _BV * _L + u * _L
                    plsc.store_compressed(cidx.at[pl.ds(offs[u], _L)],
                                          iota + (base + off), mask=msks[u])
                return (offs[_BV], m_f)

            return lax.cond(nhit, app, lambda c: c, (cnt, m_f))

        cnt, _ = lax.fori_loop(0, _NB, coll, (jnp.int32(0), m_f0))

        pad(cnt)
        nv = (cnt + _L - 1) // _L
        build_keys(nv)
        t_sv = buffer_descent(nv)

        def gcount(i, cv):
            sv = kbuf[pl.ds(i * _L, _L)]
            return cv + (sv > t_sv).astype(jnp.int32)

        need = _K - jnp.sum(lax.fori_loop(0, nv, gcount, ivzero))

        def ibit(bi, cut):
            bit = lax.shift_left(jnp.int32(1), 14 - bi)
            tmp = cut + bit - 1

            def cb(i, cv):
                ix = cidx[pl.ds(i * _L, _L)]
                sv = kbuf[pl.ds(i * _L, _L)]
                hit = (sv == t_sv) & ((ix - base) <= tmp)
                return cv + hit.astype(jnp.int32)

            cv = lax.fori_loop(0, nv, cb, ivzero)
            return jnp.where(jnp.sum(cv) >= need, cut, cut + bit)

        idx_star = lax.fori_loop(0, 15, ibit, jnp.int32(0))
        return nv, t_sv, idx_star

    def scatter_row(base, nv, t_sv, idx_star):
        def scat(i, kc):
            ix = cidx[pl.ds(i * _L, _L)]
            v = gv(ix)
            sv = kbuf[pl.ds(i * _L, _L)]
            ixo = ix - base
            keep = (sv > t_sv) | ((sv == t_sv) & (ixo <= idx_star))
            plsc.store_scatter(zero_buf, [ixo], v, mask=keep)
            plsc.store_compressed(kidx.at[pl.ds(kc, _L)], ixo, mask=keep)
            return kc + _popcnt(keep)

        lax.fori_loop(0, nv, scat, jnp.int32(0))

    def unscatter_prev():
        for u in range(_K // _L):
            ixo = kidx[pl.ds(u * _L, _L)]
            plsc.store_scatter(zero_buf, [ixo], zvec)

    out_prev = None
    for rr in range(_RPW):
        base = (rr % 2) * _N
        if rr + 1 < _RPW:
            nxt = (rr + 1) % 2
            pending_in[rr + 1] = pltpu.async_copy(
                x_hbm.at[r0 + rr + 1], rbuf.at[pl.ds(nxt * _N, _N)],
                sins[nxt])
        pending_in[rr].wait()
        nv, t_sv, idx_star = process_row(base)
        if out_prev is not None:
            out_prev.wait()
            unscatter_prev()
        scatter_row(base, nv, t_sv, idx_star)
        out_prev = pltpu.async_copy(zero_buf, o_hbm.at[r0 + rr], sout)
    out_prev.wait()


def _sc_kernel(x):
    f = pl.kernel(
        _sc_body,
        out_type=jax.ShapeDtypeStruct((_ROWS, _N), jnp.float32),
        mesh=plsc.VectorSubcoreMesh(core_axis_name="c", subcore_axis_name="s",
                                    num_cores=_NC, num_subcores=_NS),
        scratch_types=[
            pltpu.VMEM((2 * _N + _L,), jnp.float32),
            pltpu.VMEM((_N,), jnp.float32),
            pltpu.SMEM((_NB,), jnp.float32),
            pltpu.VMEM((_CAP + _L,), jnp.int32),
            pltpu.VMEM((_CAP + _L,), jnp.int32),
            pltpu.VMEM((_K + _L,), jnp.int32),
            pltpu.SemaphoreType.DMA,
            pltpu.SemaphoreType.DMA,
            pltpu.SemaphoreType.DMA,
        ],
        compiler_params=pltpu.CompilerParams(needs_layout_passes=False),
    )
    return f(x)


@jax.jit
def kernel(x):
    return _sc_kernel(x)
